# baseline (speedup 1.0000x reference)
import functools

import jax
import jax.numpy as jnp
from jax import lax
from jax.experimental import pallas as pl
from jax.experimental.pallas import tpu as pltpu
from jax.experimental.pallas import tpu_sc as plsc

BATCH = 16384
FEAT = 2048
NC = 2
NS = 16
L = 16
NW = NC * NS
ROWS_PER_W = BATCH // NW
RBLK = 8
NBLK = ROWS_PER_W // RBLK
NCHUNK = FEAT // L

_mesh = plsc.VectorSubcoreMesh(
    core_axis_name="c", subcore_axis_name="s", num_cores=NC, num_subcores=NS
)


@functools.partial(
    pl.kernel,
    out_type=jax.ShapeDtypeStruct((BATCH * FEAT,), jnp.float32),
    mesh=_mesh,
    compiler_params=pltpu.CompilerParams(needs_layout_passes=False),
    scratch_types=[
        pltpu.VMEM((FEAT,), jnp.int32),
        pltpu.VMEM((RBLK * FEAT,), jnp.float32),
        pltpu.VMEM((RBLK * FEAT,), jnp.float32),
    ],
)
def _permute(in_hbm, perm_hbm, out_hbm, perm_v, in_v, out_v):
    wid = lax.axis_index("s") * NC + lax.axis_index("c")
    base_w = wid * ROWS_PER_W
    pltpu.sync_copy(perm_hbm, perm_v)

    def block_body(b, carry):
        base = (base_w + b * RBLK) * FEAT
        pltpu.sync_copy(in_hbm.at[pl.ds(base, RBLK * FEAT)], in_v)

        def chunk_body(j, carry2):
            col = j * L
            idx = perm_v[pl.ds(col, L)]
            for r in range(RBLK):
                out_v[pl.ds(r * FEAT + col, L)] = plsc.load_gather(
                    in_v, [idx + r * FEAT]
                )
            return carry2

        lax.fori_loop(0, NCHUNK, chunk_body, 0)
        pltpu.sync_copy(out_v, out_hbm.at[pl.ds(base, RBLK * FEAT)])
        return carry

    lax.fori_loop(0, NBLK, block_body, 0)


def kernel(inputs, permutation):
    out = _permute(
        inputs.reshape(BATCH * FEAT), permutation.astype(jnp.int32)
    )
    return (out.reshape(BATCH, FEAT), 0)


# speedup vs baseline: 1.2110x; 1.2110x over previous
import functools

import jax
import jax.numpy as jnp
from jax import lax
from jax.experimental import pallas as pl
from jax.experimental.pallas import tpu as pltpu
from jax.experimental.pallas import tpu_sc as plsc

BATCH = 16384
FEAT = 2048
NC = 2
NS = 16
L = 16
NW = NC * NS
ROWS_PER_W = BATCH // NW
RBLK = 8
NBLK = ROWS_PER_W // RBLK
NCHUNK = FEAT // L

_mesh = plsc.VectorSubcoreMesh(
    core_axis_name="c", subcore_axis_name="s", num_cores=NC, num_subcores=NS
)


@functools.partial(
    pl.kernel,
    out_type=jax.ShapeDtypeStruct((BATCH * FEAT,), jnp.float32),
    mesh=_mesh,
    compiler_params=pltpu.CompilerParams(needs_layout_passes=False),
    scratch_types=[
        pltpu.VMEM((FEAT,), jnp.int32),
        pltpu.VMEM((RBLK * FEAT,), jnp.float32),
        pltpu.VMEM((RBLK * FEAT,), jnp.float32),
        pltpu.VMEM((RBLK * FEAT,), jnp.float32),
        pltpu.VMEM((RBLK * FEAT,), jnp.float32),
        pltpu.SemaphoreType.DMA,
        pltpu.SemaphoreType.DMA,
        pltpu.SemaphoreType.DMA,
        pltpu.SemaphoreType.DMA,
    ],
)
def _permute(in_hbm, perm_hbm, out_hbm, perm_v, in_v0, in_v1, out_v0, out_v1,
             sem_in0, sem_in1, sem_out0, sem_out1):
    wid = lax.axis_index("s") * NC + lax.axis_index("c")
    base_w = wid * ROWS_PER_W
    pltpu.sync_copy(perm_hbm, perm_v)
    in_bufs = (in_v0, in_v1)
    out_bufs = (out_v0, out_v1)
    sems_in = (sem_in0, sem_in1)
    sems_out = (sem_out0, sem_out1)

    def start_in(b):
        base = (base_w + b * RBLK) * FEAT
        return pltpu.async_copy(
            in_hbm.at[pl.ds(base, RBLK * FEAT)], in_bufs[b % 2], sems_in[b % 2]
        )

    def start_out(b):
        base = (base_w + b * RBLK) * FEAT
        return pltpu.async_copy(
            out_bufs[b % 2], out_hbm.at[pl.ds(base, RBLK * FEAT)],
            sems_out[b % 2],
        )

    def compute(b):
        src = in_bufs[b % 2]
        dst = out_bufs[b % 2]

        def chunk_body(j, carry2):
            col = j * L
            idx = perm_v[pl.ds(col, L)]
            for r in range(RBLK):
                dst[pl.ds(r * FEAT + col, L)] = plsc.load_gather(
                    src, [idx + r * FEAT]
                )
            return carry2

        lax.fori_loop(0, NCHUNK, chunk_body, 0)

    in_descs = [None] * NBLK
    out_descs = [None] * NBLK
    in_descs[0] = start_in(0)
    for b in range(NBLK):
        in_descs[b].wait()
        if b + 1 < NBLK:
            in_descs[b + 1] = start_in(b + 1)
        if b >= 2:
            out_descs[b - 2].wait()
        compute(b)
        out_descs[b] = start_out(b)
    out_descs[NBLK - 2].wait()
    out_descs[NBLK - 1].wait()


def kernel(inputs, permutation):
    out = _permute(
        inputs.reshape(BATCH * FEAT), permutation.astype(jnp.int32)
    )
    return (out.reshape(BATCH, FEAT), 0)


# speedup vs baseline: 2.0345x; 1.6799x over previous
import functools

import jax
import jax.numpy as jnp
from jax import lax
from jax.experimental import pallas as pl
from jax.experimental.pallas import tpu as pltpu
from jax.experimental.pallas import tpu_sc as plsc

BATCH = 16384
FEAT = 2048
NC = 2
NS = 16
L = 16
NW = NC * NS
ROWS_PER_W = BATCH // NW
RBLK = 8
NBLK = ROWS_PER_W // RBLK
NCHUNK = FEAT // L

_mesh = plsc.VectorSubcoreMesh(
    core_axis_name="c", subcore_axis_name="s", num_cores=NC, num_subcores=NS
)


@functools.partial(
    pl.kernel,
    out_type=jax.ShapeDtypeStruct((BATCH * FEAT,), jnp.float32),
    mesh=_mesh,
    compiler_params=pltpu.CompilerParams(needs_layout_passes=False),
    scratch_types=[
        pltpu.VMEM((FEAT,), jnp.int32),
        pltpu.VMEM((RBLK * FEAT,), jnp.float32),
        pltpu.VMEM((RBLK * FEAT,), jnp.float32),
        pltpu.VMEM((RBLK * FEAT,), jnp.float32),
        pltpu.VMEM((RBLK * FEAT,), jnp.float32),
        pltpu.SemaphoreType.DMA,
        pltpu.SemaphoreType.DMA,
        pltpu.SemaphoreType.DMA,
        pltpu.SemaphoreType.DMA,
    ],
)
def _permute(in_hbm, perm_hbm, out_hbm, perm_v, in_v0, in_v1, out_v0, out_v1,
             sem_in0, sem_in1, sem_out0, sem_out1):
    wid = lax.axis_index("s") * NC + lax.axis_index("c")
    base_w = wid * ROWS_PER_W
    pltpu.sync_copy(perm_hbm, perm_v)
    in_bufs = (in_v0, in_v1)
    out_bufs = (out_v0, out_v1)
    sems_in = (sem_in0, sem_in1)
    sems_out = (sem_out0, sem_out1)

    BLKE = RBLK * FEAT

    def in_desc(b, k):
        base = (base_w + b * RBLK) * FEAT
        return pltpu.make_async_copy(
            in_hbm.at[pl.ds(base, BLKE)], in_bufs[k], sems_in[k]
        )

    def out_desc(b, k):
        base = (base_w + b * RBLK) * FEAT
        return pltpu.make_async_copy(
            out_bufs[k], out_hbm.at[pl.ds(base, BLKE)], sems_out[k]
        )

    in_desc(0, 0).start()
    in_desc(1, 1).start()

    @pl.loop(0, NBLK, step=2)
    def outer(b):
        for k in range(2):
            bb = b + k
            in_desc(bb, k).wait()
            src = in_bufs[k]
            dst = out_bufs[k]

            @pl.when(bb >= 2)
            def _wait_out():
                out_desc(bb - 2, k).wait()

            @plsc.parallel_loop(0, NCHUNK, unroll=4)
            def chunk_body(j):
                col = j * L
                idx = perm_v[pl.ds(col, L)]
                for r in range(RBLK):
                    dst[pl.ds(r * FEAT + col, L)] = plsc.load_gather(
                        src, [idx + r * FEAT]
                    )

            out_desc(bb, k).start()

            @pl.when(bb + 2 < NBLK)
            def _prefetch():
                in_desc(bb + 2, k).start()

    out_desc(NBLK - 2, 0).wait()
    out_desc(NBLK - 1, 1).wait()


def kernel(inputs, permutation):
    out = _permute(
        inputs.reshape(BATCH * FEAT), permutation.astype(jnp.int32)
    )
    return (out.reshape(BATCH, FEAT), 0)


# speedup vs baseline: 5.8538x; 2.8774x over previous
import functools

import jax
import jax.numpy as jnp
from jax import lax
from jax.experimental import pallas as pl
from jax.experimental.pallas import tpu as pltpu
from jax.experimental.pallas import tpu_sc as plsc

BATCH = 16384
FEAT = 2048
NC = 2
NS = 16
L = 16
NW = NC * NS
ROWS_PER_W = BATCH // NW
RBLK = 8
NBLK = ROWS_PER_W // RBLK
NCHUNK = FEAT // L
LANE = 128
SUB = 8
TILE_WORDS = LANE * SUB

_mesh = plsc.VectorSubcoreMesh(
    core_axis_name="c", subcore_axis_name="s", num_cores=NC, num_subcores=NS
)


@functools.partial(
    pl.kernel,
    out_type=jax.ShapeDtypeStruct((BATCH, FEAT), jnp.float32),
    mesh=_mesh,
    compiler_params=pltpu.CompilerParams(
        needs_layout_passes=False, use_tc_tiling_on_sc=True
    ),
    scratch_types=[
        pltpu.VMEM((FEAT,), jnp.int32),
        pltpu.VMEM((FEAT,), jnp.int32),
        pltpu.VMEM((RBLK, FEAT), jnp.float32),
        pltpu.VMEM((RBLK, FEAT), jnp.float32),
        pltpu.VMEM((RBLK, FEAT), jnp.float32),
        pltpu.VMEM((RBLK, FEAT), jnp.float32),
        pltpu.SemaphoreType.DMA,
        pltpu.SemaphoreType.DMA,
        pltpu.SemaphoreType.DMA,
        pltpu.SemaphoreType.DMA,
    ],
)
def _permute(in_hbm, perm_hbm, out_hbm, perm_v, pcol_v,
             in_v0, in_v1, out_v0, out_v1,
             sem_in0, sem_in1, sem_out0, sem_out1):
    wid = lax.axis_index("s") * NC + lax.axis_index("c")
    base_w = wid * ROWS_PER_W
    pltpu.sync_copy(perm_hbm, perm_v)
    in_bufs = (in_v0, in_v1)
    out_bufs = (out_v0, out_v1)
    sems_in = (sem_in0, sem_in1)
    sems_out = (sem_out0, sem_out1)
    BLKE = RBLK * FEAT

    @pl.loop(0, NCHUNK)
    def _precompute(j):
        p = perm_v[pl.ds(j * L, L)]
        pcol_v[pl.ds(j * L, L)] = (
            lax.shift_left(lax.shift_right_logical(p, 7), 10)
            + lax.bitwise_and(p, LANE - 1)
        )

    def in_desc(b, k):
        base = base_w + b * RBLK
        return pltpu.make_async_copy(
            in_hbm.at[pl.ds(base, RBLK)], in_bufs[k], sems_in[k]
        )

    def out_desc(b, k):
        base = base_w + b * RBLK
        return pltpu.make_async_copy(
            out_bufs[k], out_hbm.at[pl.ds(base, RBLK)], sems_out[k]
        )

    in_desc(0, 0).start()
    in_desc(1, 1).start()

    @pl.loop(0, NBLK, step=2)
    def outer(b):
        for k in range(2):
            bb = b + k
            in_desc(bb, k).wait()
            src = in_bufs[k]
            dst = out_bufs[k]

            @pl.when(bb >= 2)
            def _wait_out():
                out_desc(bb - 2, k).wait()

            @plsc.parallel_loop(0, NCHUNK, unroll=4)
            def chunk_body(j):
                col = j * L
                idx = perm_v[pl.ds(col, L)]
                for r in range(RBLK):
                    rvec = jnp.full((L,), r, jnp.int32)
                    dst[r, pl.ds(col, L)] = plsc.load_gather(src, [rvec, idx])

            out_desc(bb, k).start()

            @pl.when(bb + 2 < NBLK)
            def _prefetch():
                in_desc(bb + 2, k).start()

    out_desc(NBLK - 2, 0).wait()
    out_desc(NBLK - 1, 1).wait()


def kernel(inputs, permutation):
    out = _permute(inputs, permutation.astype(jnp.int32))
    return (out, 0)


# speedup vs baseline: 5.8724x; 1.0032x over previous
import functools

import jax
import jax.numpy as jnp
from jax import lax
from jax.experimental import pallas as pl
from jax.experimental.pallas import tpu as pltpu
from jax.experimental.pallas import tpu_sc as plsc

BATCH = 16384
FEAT = 2048
NC = 2
NS = 16
L = 16
NW = NC * NS
ROWS_PER_W = BATCH // NW
RBLK = 8
NBLK = ROWS_PER_W // RBLK
NCHUNK = FEAT // L
LANE = 128
SUB = 8
TILE_WORDS = LANE * SUB

_mesh = plsc.VectorSubcoreMesh(
    core_axis_name="c", subcore_axis_name="s", num_cores=NC, num_subcores=NS
)


@functools.partial(
    pl.kernel,
    out_type=jax.ShapeDtypeStruct((BATCH, FEAT), jnp.float32),
    mesh=_mesh,
    compiler_params=pltpu.CompilerParams(
        needs_layout_passes=False, use_tc_tiling_on_sc=True
    ),
    scratch_types=[
        pltpu.VMEM((FEAT,), jnp.int32),
        pltpu.VMEM((RBLK, FEAT), jnp.float32),
        pltpu.VMEM((RBLK, FEAT), jnp.float32),
        pltpu.VMEM((RBLK, FEAT), jnp.float32),
        pltpu.VMEM((RBLK, FEAT), jnp.float32),
        pltpu.SemaphoreType.DMA,
        pltpu.SemaphoreType.DMA,
        pltpu.SemaphoreType.DMA,
        pltpu.SemaphoreType.DMA,
    ],
)
def _permute(in_hbm, perm_hbm, out_hbm, perm_v,
             in_v0, in_v1, out_v0, out_v1,
             sem_in0, sem_in1, sem_out0, sem_out1):
    wid = lax.axis_index("s") * NC + lax.axis_index("c")
    base_w = wid * ROWS_PER_W
    pltpu.sync_copy(perm_hbm, perm_v)
    in_bufs = (in_v0, in_v1)
    out_bufs = (out_v0, out_v1)
    sems_in = (sem_in0, sem_in1)
    sems_out = (sem_out0, sem_out1)

    def in_desc(b, k):
        base = base_w + b * RBLK
        return pltpu.make_async_copy(
            in_hbm.at[pl.ds(base, RBLK)], in_bufs[k], sems_in[k]
        )

    def out_desc(b, k):
        base = base_w + b * RBLK
        return pltpu.make_async_copy(
            out_bufs[k], out_hbm.at[pl.ds(base, RBLK)], sems_out[k]
        )

    in_desc(0, 0).start()
    in_desc(1, 1).start()

    @pl.loop(0, NBLK, step=2)
    def outer(b):
        for k in range(2):
            bb = b + k
            in_desc(bb, k).wait()
            src = in_bufs[k]
            dst = out_bufs[k]

            @pl.when(bb >= 2)
            def _wait_out():
                out_desc(bb - 2, k).wait()

            @plsc.parallel_loop(0, NCHUNK, unroll=8)
            def chunk_body(j):
                col = j * L
                idx = perm_v[pl.ds(col, L)]
                for r in range(RBLK):
                    rvec = jnp.full((L,), r, jnp.int32)
                    dst[r, pl.ds(col, L)] = plsc.load_gather(src, [rvec, idx])

            out_desc(bb, k).start()

            @pl.when(bb + 2 < NBLK)
            def _prefetch():
                in_desc(bb + 2, k).start()

    out_desc(NBLK - 2, 0).wait()
    out_desc(NBLK - 1, 1).wait()


def kernel(inputs, permutation):
    out = _permute(inputs, permutation.astype(jnp.int32))
    return (out, 0)
